# baseline (speedup 1.0000x reference)
import functools

import jax
import jax.numpy as jnp
from jax import lax
from jax.experimental import pallas as pl
from jax.experimental.pallas import tpu as pltpu
from jax.experimental.pallas import tpu_sc as plsc

N = 10000
E = 320000
IN_DIM = 128
D = 64
H3 = 3 * D
NCLS = 10
K = 5000

NC = 2
NS = 16
NW = NC * NS
EPW = E // NW
CH = 128
NCHUNK = 80
EPAD = CH * NCHUNK
RPT = N // NS
LANE = 16
NPAD = N + LANE

_mesh = plsc.VectorSubcoreMesh(
    core_axis_name="c", subcore_axis_name="s", num_cores=NC, num_subcores=NS)

f32 = jnp.float32
i32 = jnp.int32
u32 = jnp.uint32


@functools.partial(
    pl.kernel,
    out_type=(jax.ShapeDtypeStruct((NW, N), f32),
              jax.ShapeDtypeStruct((NW, N), f32)),
    mesh=_mesh,
    compiler_params=pltpu.CompilerParams(needs_layout_passes=False, use_tc_tiling_on_sc=False),
    scratch_types=[
        pltpu.VMEM((EPW,), i32),
        pltpu.VMEM((EPW,), i32),
        pltpu.VMEM((N,), f32),
        pltpu.VMEM((N,), f32),
    ],
)
def _sc_degrees(src_hbm, dst_hbm, outs_hbm, outd_hbm, src_v, dst_v, hs_v, hd_v):
    cid = lax.axis_index("c")
    sid = lax.axis_index("s")
    w = sid * NC + cid
    pltpu.sync_copy(src_hbm.at[w], src_v)
    pltpu.sync_copy(dst_hbm.at[w], dst_v)
    zero = jnp.zeros((LANE,), f32)

    def zbody(i, _):
        hs_v[pl.ds(i * LANE, LANE)] = zero
        hd_v[pl.ds(i * LANE, LANE)] = zero
        return 0

    lax.fori_loop(0, N // LANE, zbody, 0)
    one = jnp.ones((LANE,), f32)

    def body(i, _):
        s16 = src_v[pl.ds(i * LANE, LANE)]
        d16 = dst_v[pl.ds(i * LANE, LANE)]
        plsc.addupdate_scatter(hs_v, [s16], one)
        plsc.addupdate_scatter(hd_v, [d16], one)
        return 0

    lax.fori_loop(0, EPW // LANE, body, 0)
    pltpu.sync_copy(hs_v, outs_hbm.at[w])
    pltpu.sync_copy(hd_v, outd_hbm.at[w])


@functools.partial(
    pl.kernel,
    out_type=jax.ShapeDtypeStruct((NC, N, D), f32),
    mesh=_mesh,
    compiler_params=pltpu.CompilerParams(needs_layout_passes=False, use_tc_tiling_on_sc=False),
    scratch_types=[
        pltpu.VMEM((NCHUNK, CH), i32),
        pltpu.VMEM((NCHUNK, CH), i32),
        pltpu.VMEM((4 * CH, D), f32),
        pltpu.VMEM((125, D), f32),
        pltpu.VMEM_SHARED((N, D), f32),
        pltpu.SemaphoreType.DMA((4,)),
        pltpu.SemaphoreType.DMA((4,)),
    ],
)
def _sc_aggregate(xn_hbm, srcp_hbm, dstp_hbm, out_hbm,
                  si_v, di_v, rows_v, zer_v, acc_sh, gsem, ssem):
    cid = lax.axis_index("c")
    sid = lax.axis_index("s")
    w = sid * NC + cid
    zero = jnp.zeros((LANE,), f32)
    pltpu.async_copy(srcp_hbm.at[w], si_v, gsem.at[0])
    pltpu.async_copy(dstp_hbm.at[w], di_v, gsem.at[0])

    def zbody(r, _):
        for c in range(D // LANE):
            zer_v[r, pl.ds(c * LANE, LANE)] = zero
        return 0

    lax.fori_loop(0, 125, zbody, 0)
    base = sid * RPT
    for t in range(5):
        pltpu.async_copy(zer_v, acc_sh.at[pl.ds(base + t * 125, 125)], gsem.at[1])
    pltpu.make_async_copy(srcp_hbm.at[w], si_v, gsem.at[0]).wait()
    pltpu.make_async_copy(dstp_hbm.at[w], di_v, gsem.at[0]).wait()
    for t in range(5):
        pltpu.make_async_copy(zer_v, acc_sh.at[pl.ds(base + t * 125, 125)],
                              gsem.at[1]).wait()
    plsc.subcore_barrier()

    pltpu.async_copy(xn_hbm.at[si_v.at[0]], rows_v.at[pl.ds(0, CH)], gsem.at[0])

    def chunk(g, _):
        b = lax.rem(g, 4)
        nb = lax.rem(g + 1, 4)

        @pl.when(g >= 3)
        def _drain_old_scatter():
            pltpu.make_async_copy(rows_v.at[pl.ds(nb * CH, CH)],
                                  acc_sh.at[di_v.at[g - 3]], ssem.at[nb]).wait()

        @pl.when(g + 1 < NCHUNK)
        def _start_next_gather():
            pltpu.async_copy(xn_hbm.at[si_v.at[g + 1]],
                             rows_v.at[pl.ds(nb * CH, CH)], gsem.at[nb])

        pltpu.make_async_copy(xn_hbm.at[si_v.at[g]],
                              rows_v.at[pl.ds(b * CH, CH)], gsem.at[b]).wait()
        pltpu.async_copy(rows_v.at[pl.ds(b * CH, CH)],
                         acc_sh.at[di_v.at[g]], ssem.at[b], add=True)
        return 0

    lax.fori_loop(0, NCHUNK, chunk, 0)
    for t in range(NCHUNK - 3, NCHUNK):
        tb = t % 4
        pltpu.make_async_copy(rows_v.at[pl.ds(tb * CH, CH)],
                              acc_sh.at[di_v.at[t]], ssem.at[tb]).wait()
    plsc.subcore_barrier()
    pltpu.sync_copy(acc_sh.at[pl.ds(base, RPT)],
                    out_hbm.at[cid, pl.ds(base, RPT)])


@functools.partial(
    pl.kernel,
    out_type=jax.ShapeDtypeStruct((NW, N), f32),
    mesh=_mesh,
    compiler_params=pltpu.CompilerParams(needs_layout_passes=False, use_tc_tiling_on_sc=False),
    scratch_types=[
        pltpu.VMEM((N,), f32),
        pltpu.VMEM((EPW,), i32),
        pltpu.VMEM((EPW,), i32),
        pltpu.VMEM((N,), f32),
    ],
)
def _sc_scalar_agg(xs_hbm, src_hbm, dst_hbm, out_hbm, xs_v, src_v, dst_v, acc_v):
    cid = lax.axis_index("c")
    sid = lax.axis_index("s")
    w = sid * NC + cid
    pltpu.sync_copy(xs_hbm, xs_v)
    pltpu.sync_copy(src_hbm.at[w], src_v)
    pltpu.sync_copy(dst_hbm.at[w], dst_v)
    zero = jnp.zeros((LANE,), f32)

    def zbody(i, _):
        acc_v[pl.ds(i * LANE, LANE)] = zero
        return 0

    lax.fori_loop(0, N // LANE, zbody, 0)

    def body(i, _):
        s16 = src_v[pl.ds(i * LANE, LANE)]
        vals = plsc.load_gather(xs_v, [s16])
        d16 = dst_v[pl.ds(i * LANE, LANE)]
        plsc.addupdate_scatter(acc_v, [d16], vals)
        return 0

    lax.fori_loop(0, EPW // LANE, body, 0)
    pltpu.sync_copy(acc_v, out_hbm.at[w])


def _tc_prep_body(degs_ref, degd_ref, x_ref, w_ref, xn_ref, ns_ref, nd_ref):
    ds_ = jnp.sum(degs_ref[...], axis=0, keepdims=True)
    dd_ = jnp.sum(degd_ref[...], axis=0, keepdims=True)
    norms = lax.rsqrt(jnp.maximum(jnp.concatenate([ds_, dd_], axis=0), 1.0))
    ncols = jnp.transpose(norms)
    ns = ncols[:, 0:1]
    ns_ref[...] = ns
    nd_ref[...] = ncols[:, 1:2]
    xw = jnp.dot(x_ref[...], w_ref[...], preferred_element_type=f32)
    xn_ref[0:N, :] = xw * ns
    xn_ref[N:NPAD, :] = jnp.zeros((LANE, D), f32)


_tc_prep = pl.pallas_call(
    _tc_prep_body,
    out_shape=(jax.ShapeDtypeStruct((NPAD, D), f32),
               jax.ShapeDtypeStruct((N, 1), f32),
               jax.ShapeDtypeStruct((N, 1), f32)),
)


def _tc_mid_body(accp_ref, nd_ref, ns_ref, b_ref, w_ref, out_ref, xn_ref):
    agg = accp_ref[0] + accp_ref[1]
    o = jnp.maximum(agg * nd_ref[...] + b_ref[...], 0.0)
    out_ref[...] = o
    xw = jnp.dot(o, w_ref[...], preferred_element_type=f32)
    xn_ref[0:N, :] = xw * ns_ref[...]
    xn_ref[N:NPAD, :] = jnp.zeros((LANE, D), f32)


_tc_mid = pl.pallas_call(
    _tc_mid_body,
    out_shape=(jax.ShapeDtypeStruct((N, D), f32),
               jax.ShapeDtypeStruct((NPAD, D), f32)),
)


def _tc_post_body(accp_ref, nd_ref, ns_ref, b3_ref, out1_ref, out2_ref, ws_ref,
                  wn1_ref, bn1_ref, wn2_ref, bn2_ref, wn3_ref, bn3_ref,
                  out3_ref, np_ref, xs_ref):
    agg = accp_ref[0] + accp_ref[1]
    o3c = jnp.maximum(agg * nd_ref[...] + b3_ref[...], 0.0)
    out3 = jnp.concatenate([out1_ref[...], out2_ref[...], o3c], axis=1)
    out3_ref[...] = out3
    h = jnp.maximum(
        jnp.dot(out3, wn1_ref[...], preferred_element_type=f32) + bn1_ref[...], 0.0)
    h = jnp.maximum(
        jnp.dot(h, wn2_ref[...], preferred_element_type=f32) + bn2_ref[...], 0.0)
    np_ref[...] = jnp.dot(h, wn3_ref[...], preferred_element_type=f32) + bn3_ref[...]
    xs_ref[...] = jnp.dot(out3, ws_ref[...], preferred_element_type=f32) * ns_ref[...]


_tc_post = pl.pallas_call(
    _tc_post_body,
    out_shape=(jax.ShapeDtypeStruct((N, H3), f32),
               jax.ShapeDtypeStruct((N, NCLS), f32),
               jax.ShapeDtypeStruct((N, 1), f32)),
)


def _tc_finale_body(saccp_ref, nd_ref, bs_ref, out3_ref,
                    wg1_ref, bg1_ref, wg2_ref, bg2_ref, wg3_ref, bg3_ref,
                    scores_ref, scoresc_ref, hgc_ref, nsc_ref):
    srow = jnp.sum(saccp_ref[...], axis=0, keepdims=True)
    score = jnp.transpose(srow) * nd_ref[...] + bs_ref[0, 0]
    mx = jnp.max(score)
    ex = jnp.exp(score - mx)
    nsc_ref[...] = ex / jnp.sum(ex)
    ub = lax.bitcast_convert_type(score, u32)
    key = jnp.where(ub >= u32(0x80000000), ~ub, ub + u32(0x80000000))

    def sbit(i, t):
        cand = t | lax.shift_left(u32(1), u32(31) - i.astype(u32))
        cnt = jnp.sum((key >= cand).astype(i32))
        return jnp.where(cnt >= K, cand, t)

    thr = lax.fori_loop(0, 32, sbit, u32(0))
    gt = key > thr
    eq = key == thr
    need = K - jnp.sum(gt.astype(i32))
    idx = lax.broadcasted_iota(i32, (N, 1), 0)

    def ibit(i, v):
        cand = v | lax.shift_left(i32(1), i32(13) - i)
        cnt = jnp.sum((eq & (idx < cand)).astype(i32))
        return jnp.where(cnt <= need - 1, cand, v)

    istar = lax.fori_loop(0, 14, ibit, i32(0))
    mask = gt | (eq & (idx <= istar))
    w = jnp.tanh(score)
    out3 = out3_ref[...]
    vdis = out3 * jnp.where(mask, w, 0.0)
    vcom = out3 * jnp.where(mask, 0.0, w)
    ninf = f32(-jnp.inf)
    dis_mean = jnp.sum(vdis, axis=0, keepdims=True) / f32(K)
    dis_max = jnp.max(jnp.where(mask, vdis, ninf), axis=0, keepdims=True)
    com_mean = jnp.sum(vcom, axis=0, keepdims=True) / f32(N - K)
    com_max = jnp.max(jnp.where(mask, ninf, vcom), axis=0, keepdims=True)
    hg = jnp.concatenate([dis_mean, dis_max], axis=1)
    hgc = jnp.concatenate([com_mean, com_max], axis=1)
    hgc_ref[...] = hgc

    def mlp(x):
        h = jnp.maximum(
            jnp.dot(x, wg1_ref[...], preferred_element_type=f32) + bg1_ref[...], 0.0)
        h = jnp.maximum(
            jnp.dot(h, wg2_ref[...], preferred_element_type=f32) + bg2_ref[...], 0.0)
        return jnp.dot(h, wg3_ref[...], preferred_element_type=f32) + bg3_ref[...]

    scores_ref[...] = mlp(hg)
    scoresc_ref[...] = mlp(hgc)


_tc_finale = pl.pallas_call(
    _tc_finale_body,
    out_shape=(jax.ShapeDtypeStruct((1, NCLS), f32),
               jax.ShapeDtypeStruct((1, NCLS), f32),
               jax.ShapeDtypeStruct((1, 2 * H3), f32),
               jax.ShapeDtypeStruct((N, 1), f32)),
)


def kernel(feature, edge_index, label, W1, b1, W2, b2, W3, b3, Ws, bs,
           Wg1, bg1, Wg2, bg2, Wg3, bg3, Wn1, bn1, Wn2, bn2, Wn3, bn3):
    del label
    src2d = edge_index[0].reshape(NW, EPW)
    dst2d = edge_index[1].reshape(NW, EPW)
    spad = jnp.full((NW, EPAD - EPW), N, dtype=i32)
    dpad = jnp.zeros((NW, EPAD - EPW), dtype=i32)
    srcp = jnp.concatenate([src2d, spad], axis=1).reshape(NW, NCHUNK, CH)
    dstp = jnp.concatenate([dst2d, dpad], axis=1).reshape(NW, NCHUNK, CH)

    degs, degd = _sc_degrees(src2d, dst2d)
    xn1, ns, nd = _tc_prep(degs, degd, feature, W1)
    acc1 = _sc_aggregate(xn1, srcp, dstp)
    out1, xn2 = _tc_mid(acc1, nd, ns, b1, W2)
    acc2 = _sc_aggregate(xn2, srcp, dstp)
    out2, xn3 = _tc_mid(acc2, nd, ns, b2, W3)
    acc3 = _sc_aggregate(xn3, srcp, dstp)
    out3, node_pred, xs = _tc_post(acc3, nd, ns, b3, out1, out2, Ws,
                                   Wn1, bn1, Wn2, bn2, Wn3, bn3)
    sacc = _sc_scalar_agg(xs.reshape(N), src2d, dst2d)
    scores, scores_com, hgc, nsc = _tc_finale(
        sacc, nd, bs.reshape(1, 1), out3, Wg1, bg1, Wg2, bg2, Wg3, bg3)
    return (scores, scores_com, hgc, node_pred, nsc.reshape(N))


# speedup vs baseline: 1.0420x; 1.0420x over previous
import functools

import jax
import jax.numpy as jnp
from jax import lax
from jax.experimental import pallas as pl
from jax.experimental.pallas import tpu as pltpu
from jax.experimental.pallas import tpu_sc as plsc

N = 10000
E = 320000
IN_DIM = 128
D = 64
H3 = 3 * D
NCLS = 10
K = 5000

NC = 2
NS = 16
NW = NC * NS
EPW = E // NW
CH = 128
NCHUNK = 80
EPAD = CH * NCHUNK
RPT = N // NS
LANE = 16
NPAD = N + LANE

_mesh = plsc.VectorSubcoreMesh(
    core_axis_name="c", subcore_axis_name="s", num_cores=NC, num_subcores=NS)

f32 = jnp.float32
i32 = jnp.int32
u32 = jnp.uint32


@functools.partial(
    pl.kernel,
    out_type=(jax.ShapeDtypeStruct((NW, N), f32),
              jax.ShapeDtypeStruct((NW, N), f32)),
    mesh=_mesh,
    compiler_params=pltpu.CompilerParams(needs_layout_passes=False, use_tc_tiling_on_sc=False),
    scratch_types=[
        pltpu.VMEM((EPW,), i32),
        pltpu.VMEM((EPW,), i32),
        pltpu.VMEM((N,), f32),
        pltpu.VMEM((N,), f32),
    ],
)
def _sc_degrees(src_hbm, dst_hbm, outs_hbm, outd_hbm, src_v, dst_v, hs_v, hd_v):
    cid = lax.axis_index("c")
    sid = lax.axis_index("s")
    w = sid * NC + cid
    pltpu.sync_copy(src_hbm.at[w], src_v)
    pltpu.sync_copy(dst_hbm.at[w], dst_v)
    zero = jnp.zeros((LANE,), f32)

    def zbody(i, _):
        hs_v[pl.ds(i * LANE, LANE)] = zero
        hd_v[pl.ds(i * LANE, LANE)] = zero
        return 0

    lax.fori_loop(0, N // LANE, zbody, 0)
    one = jnp.ones((LANE,), f32)

    def body(i, _):
        s16 = src_v[pl.ds(i * LANE, LANE)]
        d16 = dst_v[pl.ds(i * LANE, LANE)]
        plsc.addupdate_scatter(hs_v, [s16], one)
        plsc.addupdate_scatter(hd_v, [d16], one)
        return 0

    lax.fori_loop(0, EPW // LANE, body, 0)
    pltpu.sync_copy(hs_v, outs_hbm.at[w])
    pltpu.sync_copy(hd_v, outd_hbm.at[w])


@functools.partial(
    pl.kernel,
    out_type=jax.ShapeDtypeStruct((NC, N, D), f32),
    mesh=_mesh,
    compiler_params=pltpu.CompilerParams(needs_layout_passes=False, use_tc_tiling_on_sc=False),
    scratch_types=[
        pltpu.VMEM((NCHUNK, CH), i32),
        pltpu.VMEM((NCHUNK, CH), i32),
        pltpu.VMEM((4 * CH, D), f32),
        pltpu.VMEM((125, D), f32),
        pltpu.VMEM_SHARED((N, D), f32),
        pltpu.SemaphoreType.DMA((4,)),
        pltpu.SemaphoreType.DMA((4,)),
    ],
)
def _sc_aggregate(xn_hbm, srcp_hbm, dstp_hbm, out_hbm,
                  si_v, di_v, rows_v, zer_v, acc_sh, gsem, ssem):
    cid = lax.axis_index("c")
    sid = lax.axis_index("s")
    w = sid * NC + cid
    zero = jnp.zeros((LANE,), f32)
    pltpu.async_copy(srcp_hbm.at[w], si_v, gsem.at[0])
    pltpu.async_copy(dstp_hbm.at[w], di_v, gsem.at[0])

    def zbody(r, _):
        for c in range(D // LANE):
            zer_v[r, pl.ds(c * LANE, LANE)] = zero
        return 0

    lax.fori_loop(0, 125, zbody, 0)
    base = sid * RPT
    for t in range(5):
        pltpu.async_copy(zer_v, acc_sh.at[pl.ds(base + t * 125, 125)], gsem.at[1])
    pltpu.make_async_copy(srcp_hbm.at[w], si_v, gsem.at[0]).wait()
    pltpu.make_async_copy(dstp_hbm.at[w], di_v, gsem.at[0]).wait()
    for t in range(5):
        pltpu.make_async_copy(zer_v, acc_sh.at[pl.ds(base + t * 125, 125)],
                              gsem.at[1]).wait()
    plsc.subcore_barrier()

    pltpu.async_copy(xn_hbm.at[si_v.at[0]], rows_v.at[pl.ds(0, CH)], gsem.at[0])

    def chunk(g, _):
        b = lax.rem(g, 4)
        nb = lax.rem(g + 1, 4)

        @pl.when(g >= 3)
        def _drain_old_scatter():
            pltpu.make_async_copy(rows_v.at[pl.ds(nb * CH, CH)],
                                  acc_sh.at[di_v.at[g - 3]], ssem.at[nb]).wait()

        @pl.when(g + 1 < NCHUNK)
        def _start_next_gather():
            pltpu.async_copy(xn_hbm.at[si_v.at[g + 1]],
                             rows_v.at[pl.ds(nb * CH, CH)], gsem.at[nb])

        pltpu.make_async_copy(xn_hbm.at[si_v.at[g]],
                              rows_v.at[pl.ds(b * CH, CH)], gsem.at[b]).wait()
        pltpu.async_copy(rows_v.at[pl.ds(b * CH, CH)],
                         acc_sh.at[di_v.at[g]], ssem.at[b], add=True)
        return 0

    lax.fori_loop(0, NCHUNK, chunk, 0)
    for t in range(NCHUNK - 3, NCHUNK):
        tb = t % 4
        pltpu.make_async_copy(rows_v.at[pl.ds(tb * CH, CH)],
                              acc_sh.at[di_v.at[t]], ssem.at[tb]).wait()
    plsc.subcore_barrier()
    pltpu.sync_copy(acc_sh.at[pl.ds(base, RPT)],
                    out_hbm.at[cid, pl.ds(base, RPT)])


@functools.partial(
    pl.kernel,
    out_type=jax.ShapeDtypeStruct((NW, N), f32),
    mesh=_mesh,
    compiler_params=pltpu.CompilerParams(needs_layout_passes=False, use_tc_tiling_on_sc=False),
    scratch_types=[
        pltpu.VMEM((N,), f32),
        pltpu.VMEM((EPW,), i32),
        pltpu.VMEM((EPW,), i32),
        pltpu.VMEM((N,), f32),
    ],
)
def _sc_scalar_agg(xs_hbm, src_hbm, dst_hbm, out_hbm, xs_v, src_v, dst_v, acc_v):
    cid = lax.axis_index("c")
    sid = lax.axis_index("s")
    w = sid * NC + cid
    pltpu.sync_copy(xs_hbm, xs_v)
    pltpu.sync_copy(src_hbm.at[w], src_v)
    pltpu.sync_copy(dst_hbm.at[w], dst_v)
    zero = jnp.zeros((LANE,), f32)

    def zbody(i, _):
        acc_v[pl.ds(i * LANE, LANE)] = zero
        return 0

    lax.fori_loop(0, N // LANE, zbody, 0)

    def body(i, _):
        s16 = src_v[pl.ds(i * LANE, LANE)]
        vals = plsc.load_gather(xs_v, [s16])
        d16 = dst_v[pl.ds(i * LANE, LANE)]
        plsc.addupdate_scatter(acc_v, [d16], vals)
        return 0

    lax.fori_loop(0, EPW // LANE, body, 0)
    pltpu.sync_copy(acc_v, out_hbm.at[w])


def _tc_norms_body(degs_ref, degd_ref, ns_ref, nd_ref):
    ds_ = jnp.sum(degs_ref[...], axis=0, keepdims=True)
    dd_ = jnp.sum(degd_ref[...], axis=0, keepdims=True)
    ns_ref[...] = lax.rsqrt(jnp.maximum(ds_, 1.0))
    nd_ref[...] = lax.rsqrt(jnp.maximum(dd_, 1.0))


_tc_norms = pl.pallas_call(
    _tc_norms_body,
    out_shape=(jax.ShapeDtypeStruct((1, N), f32),
               jax.ShapeDtypeStruct((1, N), f32)),
)


def _tc_prep_body(x_ref, w_ref, ns_ref, xn_ref):
    xw = jnp.dot(x_ref[...], w_ref[...], preferred_element_type=f32)
    xn_ref[0:N, :] = xw * ns_ref[...]
    xn_ref[N:NPAD, :] = jnp.zeros((LANE, D), f32)


_tc_prep = pl.pallas_call(
    _tc_prep_body,
    out_shape=jax.ShapeDtypeStruct((NPAD, D), f32),
)


def _tc_mid_body(accp_ref, nd_ref, ns_ref, b_ref, w_ref, out_ref, xn_ref):
    agg = accp_ref[0] + accp_ref[1]
    o = jnp.maximum(agg * nd_ref[...] + b_ref[...], 0.0)
    out_ref[...] = o
    xw = jnp.dot(o, w_ref[...], preferred_element_type=f32)
    xn_ref[0:N, :] = xw * ns_ref[...]
    xn_ref[N:NPAD, :] = jnp.zeros((LANE, D), f32)


_tc_mid = pl.pallas_call(
    _tc_mid_body,
    out_shape=(jax.ShapeDtypeStruct((N, D), f32),
               jax.ShapeDtypeStruct((NPAD, D), f32)),
)


def _tc_post_body(accp_ref, nd_ref, ns_ref, b3_ref, out1_ref, out2_ref, ws_ref,
                  wn1_ref, bn1_ref, wn2_ref, bn2_ref, wn3_ref, bn3_ref,
                  out3_ref, np_ref, xs_ref):
    agg = accp_ref[0] + accp_ref[1]
    o3c = jnp.maximum(agg * nd_ref[...] + b3_ref[...], 0.0)
    out3 = jnp.concatenate([out1_ref[...], out2_ref[...], o3c], axis=1)
    out3_ref[...] = out3
    h = jnp.maximum(
        jnp.dot(out3, wn1_ref[...], preferred_element_type=f32) + bn1_ref[...], 0.0)
    h = jnp.maximum(
        jnp.dot(h, wn2_ref[...], preferred_element_type=f32) + bn2_ref[...], 0.0)
    np_ref[...] = jnp.dot(h, wn3_ref[...], preferred_element_type=f32) + bn3_ref[...]
    xs_ref[...] = jnp.dot(out3, ws_ref[...], preferred_element_type=f32) * ns_ref[...]


_tc_post = pl.pallas_call(
    _tc_post_body,
    out_shape=(jax.ShapeDtypeStruct((N, H3), f32),
               jax.ShapeDtypeStruct((N, NCLS), f32),
               jax.ShapeDtypeStruct((N, 1), f32)),
)


def _tc_topk_body(saccp_ref, nd_ref, bs_ref, nsc_ref, wdis_ref, wcom_ref, md_ref):
    srow = jnp.sum(saccp_ref[...], axis=0, keepdims=True)
    score = srow * nd_ref[...] + bs_ref[0, 0]
    mx = jnp.max(score)
    ex = jnp.exp(score - mx)
    nsc_ref[...] = ex / jnp.sum(ex)
    ub = lax.bitcast_convert_type(score, u32)
    key = jnp.where(ub >= u32(0x80000000), ~ub, ub + u32(0x80000000))

    def sbit(i, t):
        cand = t | lax.shift_left(u32(1), u32(31) - i.astype(u32))
        cnt = jnp.sum((key >= cand).astype(i32))
        return jnp.where(cnt >= K, cand, t)

    thr = lax.fori_loop(0, 32, sbit, u32(0))
    gt = key > thr
    eq = key == thr
    need = K - jnp.sum(gt.astype(i32))
    idx = lax.broadcasted_iota(i32, (1, N), 1)

    def ibit(i, v):
        cand = v | lax.shift_left(i32(1), i32(13) - i)
        cnt = jnp.sum((eq & (idx < cand)).astype(i32))
        return jnp.where(cnt <= need - 1, cand, v)

    istar = lax.fori_loop(0, 14, ibit, i32(0))
    mask = gt | (eq & (idx <= istar))
    w = jnp.tanh(score)
    wdis_ref[...] = jnp.where(mask, w, 0.0)
    wcom_ref[...] = jnp.where(mask, 0.0, w)
    md_ref[...] = mask.astype(f32)


_tc_topk = pl.pallas_call(
    _tc_topk_body,
    out_shape=(jax.ShapeDtypeStruct((1, N), f32),
               jax.ShapeDtypeStruct((1, N), f32),
               jax.ShapeDtypeStruct((1, N), f32),
               jax.ShapeDtypeStruct((1, N), f32)),
)


def _tc_readout_body(out3_ref, wdis_ref, wcom_ref, md_ref,
                     wg1_ref, bg1_ref, wg2_ref, bg2_ref, wg3_ref, bg3_ref,
                     scores_ref, scoresc_ref, hgc_ref):
    out3 = out3_ref[...]
    maskb = md_ref[...] != 0.0
    vdis = out3 * wdis_ref[...]
    vcom = out3 * wcom_ref[...]
    ninf = f32(-jnp.inf)
    dis_mean = jnp.sum(vdis, axis=0, keepdims=True) / f32(K)
    dis_max = jnp.max(jnp.where(maskb, vdis, ninf), axis=0, keepdims=True)
    com_mean = jnp.sum(vcom, axis=0, keepdims=True) / f32(N - K)
    com_max = jnp.max(jnp.where(maskb, ninf, vcom), axis=0, keepdims=True)
    hg = jnp.concatenate([dis_mean, dis_max], axis=1)
    hgc = jnp.concatenate([com_mean, com_max], axis=1)
    hgc_ref[...] = hgc

    def mlp(x):
        h = jnp.maximum(
            jnp.dot(x, wg1_ref[...], preferred_element_type=f32) + bg1_ref[...], 0.0)
        h = jnp.maximum(
            jnp.dot(h, wg2_ref[...], preferred_element_type=f32) + bg2_ref[...], 0.0)
        return jnp.dot(h, wg3_ref[...], preferred_element_type=f32) + bg3_ref[...]

    scores_ref[...] = mlp(hg)
    scoresc_ref[...] = mlp(hgc)


_tc_readout = pl.pallas_call(
    _tc_readout_body,
    out_shape=(jax.ShapeDtypeStruct((1, NCLS), f32),
               jax.ShapeDtypeStruct((1, NCLS), f32),
               jax.ShapeDtypeStruct((1, 2 * H3), f32)),
)


def kernel(feature, edge_index, label, W1, b1, W2, b2, W3, b3, Ws, bs,
           Wg1, bg1, Wg2, bg2, Wg3, bg3, Wn1, bn1, Wn2, bn2, Wn3, bn3):
    del label
    src2d = edge_index[0].reshape(NW, EPW)
    dst2d = edge_index[1].reshape(NW, EPW)
    spad = jnp.full((NW, EPAD - EPW), N, dtype=i32)
    dpad = jnp.zeros((NW, EPAD - EPW), dtype=i32)
    srcp = jnp.concatenate([src2d, spad], axis=1).reshape(NW, NCHUNK, CH)
    dstp = jnp.concatenate([dst2d, dpad], axis=1).reshape(NW, NCHUNK, CH)

    degs, degd = _sc_degrees(src2d, dst2d)
    ns_row, nd_row = _tc_norms(degs, degd)
    ns = ns_row.reshape(N, 1)
    nd = nd_row.reshape(N, 1)

    xn1 = _tc_prep(feature, W1, ns)
    acc1 = _sc_aggregate(xn1, srcp, dstp)
    out1, xn2 = _tc_mid(acc1, nd, ns, b1, W2)
    acc2 = _sc_aggregate(xn2, srcp, dstp)
    out2, xn3 = _tc_mid(acc2, nd, ns, b2, W3)
    acc3 = _sc_aggregate(xn3, srcp, dstp)
    out3, node_pred, xs = _tc_post(acc3, nd, ns, b3, out1, out2, Ws,
                                   Wn1, bn1, Wn2, bn2, Wn3, bn3)
    sacc = _sc_scalar_agg(xs.reshape(N), src2d, dst2d)
    nsc_row, wdis_row, wcom_row, md_row = _tc_topk(sacc, nd_row, bs.reshape(1, 1))
    scores, scores_com, hgc = _tc_readout(
        out3, wdis_row.reshape(N, 1), wcom_row.reshape(N, 1), md_row.reshape(N, 1),
        Wg1, bg1, Wg2, bg2, Wg3, bg3)
    return (scores, scores_com, hgc, node_pred, nsc_row.reshape(N))


# speedup vs baseline: 1.0423x; 1.0003x over previous
import functools

import jax
import jax.numpy as jnp
from jax import lax
from jax.experimental import pallas as pl
from jax.experimental.pallas import tpu as pltpu
from jax.experimental.pallas import tpu_sc as plsc

N = 10000
E = 320000
IN_DIM = 128
D = 64
H3 = 3 * D
NCLS = 10
K = 5000

NC = 2
NS = 16
NW = NC * NS
EPW = E // NW
CH = 128
NCHUNK = 80
EPAD = CH * NCHUNK
RPT = N // NS
LANE = 16
NPAD = N + LANE

_mesh = plsc.VectorSubcoreMesh(
    core_axis_name="c", subcore_axis_name="s", num_cores=NC, num_subcores=NS)

f32 = jnp.float32
i32 = jnp.int32
u32 = jnp.uint32


@functools.partial(
    pl.kernel,
    out_type=(jax.ShapeDtypeStruct((NW, N), f32),
              jax.ShapeDtypeStruct((NW, N), f32)),
    mesh=_mesh,
    compiler_params=pltpu.CompilerParams(needs_layout_passes=False, use_tc_tiling_on_sc=False),
    scratch_types=[
        pltpu.VMEM((EPW,), i32),
        pltpu.VMEM((EPW,), i32),
        pltpu.VMEM((N,), f32),
        pltpu.VMEM((N,), f32),
    ],
)
def _sc_degrees(src_hbm, dst_hbm, outs_hbm, outd_hbm, src_v, dst_v, hs_v, hd_v):
    cid = lax.axis_index("c")
    sid = lax.axis_index("s")
    w = sid * NC + cid
    pltpu.sync_copy(src_hbm.at[w], src_v)
    pltpu.sync_copy(dst_hbm.at[w], dst_v)
    zero = jnp.zeros((LANE,), f32)

    def zbody(i, _):
        hs_v[pl.ds(i * LANE, LANE)] = zero
        hd_v[pl.ds(i * LANE, LANE)] = zero
        return 0

    lax.fori_loop(0, N // LANE, zbody, 0)
    one = jnp.ones((LANE,), f32)

    def body(i, _):
        s16 = src_v[pl.ds(i * LANE, LANE)]
        d16 = dst_v[pl.ds(i * LANE, LANE)]
        plsc.addupdate_scatter(hs_v, [s16], one)
        plsc.addupdate_scatter(hd_v, [d16], one)
        return 0

    lax.fori_loop(0, EPW // LANE, body, 0)
    pltpu.sync_copy(hs_v, outs_hbm.at[w])
    pltpu.sync_copy(hd_v, outd_hbm.at[w])


@functools.partial(
    pl.kernel,
    out_type=jax.ShapeDtypeStruct((NC, N, D), f32),
    mesh=_mesh,
    compiler_params=pltpu.CompilerParams(needs_layout_passes=False, use_tc_tiling_on_sc=False),
    scratch_types=[
        pltpu.VMEM((NCHUNK, CH), i32),
        pltpu.VMEM((NCHUNK, CH), i32),
        pltpu.VMEM((6 * CH, D), f32),
        pltpu.VMEM((125, D), f32),
        pltpu.VMEM_SHARED((N, D), f32),
        pltpu.SemaphoreType.DMA((6,)),
        pltpu.SemaphoreType.DMA((6,)),
    ],
)
def _sc_aggregate(xn_hbm, srcp_hbm, dstp_hbm, out_hbm,
                  si_v, di_v, rows_v, zer_v, acc_sh, gsem, ssem):
    cid = lax.axis_index("c")
    sid = lax.axis_index("s")
    w = sid * NC + cid
    zero = jnp.zeros((LANE,), f32)
    pltpu.async_copy(srcp_hbm.at[w], si_v, gsem.at[0])
    pltpu.async_copy(dstp_hbm.at[w], di_v, gsem.at[0])

    def zbody(r, _):
        for c in range(D // LANE):
            zer_v[r, pl.ds(c * LANE, LANE)] = zero
        return 0

    lax.fori_loop(0, 125, zbody, 0)
    base = sid * RPT
    for t in range(5):
        pltpu.async_copy(zer_v, acc_sh.at[pl.ds(base + t * 125, 125)], gsem.at[1])
    pltpu.make_async_copy(srcp_hbm.at[w], si_v, gsem.at[0]).wait()
    pltpu.make_async_copy(dstp_hbm.at[w], di_v, gsem.at[0]).wait()
    for t in range(5):
        pltpu.make_async_copy(zer_v, acc_sh.at[pl.ds(base + t * 125, 125)],
                              gsem.at[1]).wait()
    plsc.subcore_barrier()

    pltpu.async_copy(xn_hbm.at[si_v.at[0]], rows_v.at[pl.ds(0, CH)], gsem.at[0])

    def chunk(g, _):
        b = lax.rem(g, 6)
        nb = lax.rem(g + 1, 6)

        @pl.when(g >= 5)
        def _drain_old_scatter():
            pltpu.make_async_copy(rows_v.at[pl.ds(nb * CH, CH)],
                                  acc_sh.at[di_v.at[g - 5]], ssem.at[nb]).wait()

        @pl.when(g + 1 < NCHUNK)
        def _start_next_gather():
            pltpu.async_copy(xn_hbm.at[si_v.at[g + 1]],
                             rows_v.at[pl.ds(nb * CH, CH)], gsem.at[nb])

        pltpu.make_async_copy(xn_hbm.at[si_v.at[g]],
                              rows_v.at[pl.ds(b * CH, CH)], gsem.at[b]).wait()
        pltpu.async_copy(rows_v.at[pl.ds(b * CH, CH)],
                         acc_sh.at[di_v.at[g]], ssem.at[b], add=True)
        return 0

    lax.fori_loop(0, NCHUNK, chunk, 0)
    for t in range(NCHUNK - 5, NCHUNK):
        tb = t % 6
        pltpu.make_async_copy(rows_v.at[pl.ds(tb * CH, CH)],
                              acc_sh.at[di_v.at[t]], ssem.at[tb]).wait()
    plsc.subcore_barrier()
    pltpu.sync_copy(acc_sh.at[pl.ds(base, RPT)],
                    out_hbm.at[cid, pl.ds(base, RPT)])


@functools.partial(
    pl.kernel,
    out_type=jax.ShapeDtypeStruct((NW, N), f32),
    mesh=_mesh,
    compiler_params=pltpu.CompilerParams(needs_layout_passes=False, use_tc_tiling_on_sc=False),
    scratch_types=[
        pltpu.VMEM((N,), f32),
        pltpu.VMEM((EPW,), i32),
        pltpu.VMEM((EPW,), i32),
        pltpu.VMEM((N,), f32),
    ],
)
def _sc_scalar_agg(xs_hbm, src_hbm, dst_hbm, out_hbm, xs_v, src_v, dst_v, acc_v):
    cid = lax.axis_index("c")
    sid = lax.axis_index("s")
    w = sid * NC + cid
    pltpu.sync_copy(xs_hbm, xs_v)
    pltpu.sync_copy(src_hbm.at[w], src_v)
    pltpu.sync_copy(dst_hbm.at[w], dst_v)
    zero = jnp.zeros((LANE,), f32)

    def zbody(i, _):
        acc_v[pl.ds(i * LANE, LANE)] = zero
        return 0

    lax.fori_loop(0, N // LANE, zbody, 0)

    def body(i, _):
        s16 = src_v[pl.ds(i * LANE, LANE)]
        vals = plsc.load_gather(xs_v, [s16])
        d16 = dst_v[pl.ds(i * LANE, LANE)]
        plsc.addupdate_scatter(acc_v, [d16], vals)
        return 0

    lax.fori_loop(0, EPW // LANE, body, 0)
    pltpu.sync_copy(acc_v, out_hbm.at[w])


def _tc_norms_body(degs_ref, degd_ref, ns_ref, nd_ref):
    ds_ = jnp.sum(degs_ref[...], axis=0, keepdims=True)
    dd_ = jnp.sum(degd_ref[...], axis=0, keepdims=True)
    ns_ref[...] = lax.rsqrt(jnp.maximum(ds_, 1.0))
    nd_ref[...] = lax.rsqrt(jnp.maximum(dd_, 1.0))


_tc_norms = pl.pallas_call(
    _tc_norms_body,
    out_shape=(jax.ShapeDtypeStruct((1, N), f32),
               jax.ShapeDtypeStruct((1, N), f32)),
)


def _tc_prep_body(x_ref, w_ref, ns_ref, xn_ref):
    xw = jnp.dot(x_ref[...], w_ref[...], preferred_element_type=f32)
    xn_ref[0:N, :] = xw * ns_ref[...]
    xn_ref[N:NPAD, :] = jnp.zeros((LANE, D), f32)


_tc_prep = pl.pallas_call(
    _tc_prep_body,
    out_shape=jax.ShapeDtypeStruct((NPAD, D), f32),
)


def _tc_mid_body(accp_ref, nd_ref, ns_ref, b_ref, w_ref, out_ref, xn_ref):
    agg = accp_ref[0] + accp_ref[1]
    o = jnp.maximum(agg * nd_ref[...] + b_ref[...], 0.0)
    out_ref[...] = o
    xw = jnp.dot(o, w_ref[...], preferred_element_type=f32)
    xn_ref[0:N, :] = xw * ns_ref[...]
    xn_ref[N:NPAD, :] = jnp.zeros((LANE, D), f32)


_tc_mid = pl.pallas_call(
    _tc_mid_body,
    out_shape=(jax.ShapeDtypeStruct((N, D), f32),
               jax.ShapeDtypeStruct((NPAD, D), f32)),
)


def _tc_post_body(accp_ref, nd_ref, ns_ref, b3_ref, out1_ref, out2_ref, ws_ref,
                  wn1_ref, bn1_ref, wn2_ref, bn2_ref, wn3_ref, bn3_ref,
                  out3_ref, np_ref, xs_ref):
    agg = accp_ref[0] + accp_ref[1]
    o3c = jnp.maximum(agg * nd_ref[...] + b3_ref[...], 0.0)
    out3 = jnp.concatenate([out1_ref[...], out2_ref[...], o3c], axis=1)
    out3_ref[...] = out3
    h = jnp.maximum(
        jnp.dot(out3, wn1_ref[...], preferred_element_type=f32) + bn1_ref[...], 0.0)
    h = jnp.maximum(
        jnp.dot(h, wn2_ref[...], preferred_element_type=f32) + bn2_ref[...], 0.0)
    np_ref[...] = jnp.dot(h, wn3_ref[...], preferred_element_type=f32) + bn3_ref[...]
    xs_ref[...] = jnp.dot(out3, ws_ref[...], preferred_element_type=f32) * ns_ref[...]


_tc_post = pl.pallas_call(
    _tc_post_body,
    out_shape=(jax.ShapeDtypeStruct((N, H3), f32),
               jax.ShapeDtypeStruct((N, NCLS), f32),
               jax.ShapeDtypeStruct((N, 1), f32)),
)


def _tc_topk_body(saccp_ref, nd_ref, bs_ref, nsc_ref, wdis_ref, wcom_ref, md_ref):
    srow = jnp.sum(saccp_ref[...], axis=0, keepdims=True)
    score = srow * nd_ref[...] + bs_ref[0, 0]
    mx = jnp.max(score)
    ex = jnp.exp(score - mx)
    nsc_ref[...] = ex / jnp.sum(ex)
    ub = lax.bitcast_convert_type(score, u32)
    key = jnp.where(ub >= u32(0x80000000), ~ub, ub + u32(0x80000000))

    def sbit(i, t):
        cand = t | lax.shift_left(u32(1), u32(31) - i.astype(u32))
        cnt = jnp.sum((key >= cand).astype(i32))
        return jnp.where(cnt >= K, cand, t)

    thr = lax.fori_loop(0, 32, sbit, u32(0))
    gt = key > thr
    eq = key == thr
    need = K - jnp.sum(gt.astype(i32))
    idx = lax.broadcasted_iota(i32, (1, N), 1)

    def ibit(i, v):
        cand = v | lax.shift_left(i32(1), i32(13) - i)
        cnt = jnp.sum((eq & (idx < cand)).astype(i32))
        return jnp.where(cnt <= need - 1, cand, v)

    istar = lax.fori_loop(0, 14, ibit, i32(0))
    mask = gt | (eq & (idx <= istar))
    w = jnp.tanh(score)
    wdis_ref[...] = jnp.where(mask, w, 0.0)
    wcom_ref[...] = jnp.where(mask, 0.0, w)
    md_ref[...] = mask.astype(f32)


_tc_topk = pl.pallas_call(
    _tc_topk_body,
    out_shape=(jax.ShapeDtypeStruct((1, N), f32),
               jax.ShapeDtypeStruct((1, N), f32),
               jax.ShapeDtypeStruct((1, N), f32),
               jax.ShapeDtypeStruct((1, N), f32)),
)


def _tc_readout_body(out3_ref, wdis_ref, wcom_ref, md_ref,
                     wg1_ref, bg1_ref, wg2_ref, bg2_ref, wg3_ref, bg3_ref,
                     scores_ref, scoresc_ref, hgc_ref):
    out3 = out3_ref[...]
    maskb = md_ref[...] != 0.0
    vdis = out3 * wdis_ref[...]
    vcom = out3 * wcom_ref[...]
    ninf = f32(-jnp.inf)
    dis_mean = jnp.sum(vdis, axis=0, keepdims=True) / f32(K)
    dis_max = jnp.max(jnp.where(maskb, vdis, ninf), axis=0, keepdims=True)
    com_mean = jnp.sum(vcom, axis=0, keepdims=True) / f32(N - K)
    com_max = jnp.max(jnp.where(maskb, ninf, vcom), axis=0, keepdims=True)
    hg = jnp.concatenate([dis_mean, dis_max], axis=1)
    hgc = jnp.concatenate([com_mean, com_max], axis=1)
    hgc_ref[...] = hgc

    def mlp(x):
        h = jnp.maximum(
            jnp.dot(x, wg1_ref[...], preferred_element_type=f32) + bg1_ref[...], 0.0)
        h = jnp.maximum(
            jnp.dot(h, wg2_ref[...], preferred_element_type=f32) + bg2_ref[...], 0.0)
        return jnp.dot(h, wg3_ref[...], preferred_element_type=f32) + bg3_ref[...]

    scores_ref[...] = mlp(hg)
    scoresc_ref[...] = mlp(hgc)


_tc_readout = pl.pallas_call(
    _tc_readout_body,
    out_shape=(jax.ShapeDtypeStruct((1, NCLS), f32),
               jax.ShapeDtypeStruct((1, NCLS), f32),
               jax.ShapeDtypeStruct((1, 2 * H3), f32)),
)


def kernel(feature, edge_index, label, W1, b1, W2, b2, W3, b3, Ws, bs,
           Wg1, bg1, Wg2, bg2, Wg3, bg3, Wn1, bn1, Wn2, bn2, Wn3, bn3):
    del label
    src2d = edge_index[0].reshape(NW, EPW)
    dst2d = edge_index[1].reshape(NW, EPW)
    spad = jnp.full((NW, EPAD - EPW), N, dtype=i32)
    dpad = jnp.zeros((NW, EPAD - EPW), dtype=i32)
    srcp = jnp.concatenate([src2d, spad], axis=1).reshape(NW, NCHUNK, CH)
    dstp = jnp.concatenate([dst2d, dpad], axis=1).reshape(NW, NCHUNK, CH)

    degs, degd = _sc_degrees(src2d, dst2d)
    ns_row, nd_row = _tc_norms(degs, degd)
    ns = ns_row.reshape(N, 1)
    nd = nd_row.reshape(N, 1)

    xn1 = _tc_prep(feature, W1, ns)
    acc1 = _sc_aggregate(xn1, srcp, dstp)
    out1, xn2 = _tc_mid(acc1, nd, ns, b1, W2)
    acc2 = _sc_aggregate(xn2, srcp, dstp)
    out2, xn3 = _tc_mid(acc2, nd, ns, b2, W3)
    acc3 = _sc_aggregate(xn3, srcp, dstp)
    out3, node_pred, xs = _tc_post(acc3, nd, ns, b3, out1, out2, Ws,
                                   Wn1, bn1, Wn2, bn2, Wn3, bn3)
    sacc = _sc_scalar_agg(xs.reshape(N), src2d, dst2d)
    nsc_row, wdis_row, wcom_row, md_row = _tc_topk(sacc, nd_row, bs.reshape(1, 1))
    scores, scores_com, hgc = _tc_readout(
        out3, wdis_row.reshape(N, 1), wcom_row.reshape(N, 1), md_row.reshape(N, 1),
        Wg1, bg1, Wg2, bg2, Wg3, bg3)
    return (scores, scores_com, hgc, node_pred, nsc_row.reshape(N))
